# baseline (speedup 1.0000x reference)
import jax
import jax.numpy as jnp
from jax import lax
from jax.experimental import pallas as pl
from jax.experimental.pallas import tpu as pltpu
from jax.experimental.pallas import tpu_sc as plsc

M = 10000
B = 4096
IMG = (3, 32, 32)

NW = 32
UNITS_A = 20
UNITS_B = 19
SPLIT = 17
BASE_B = SPLIT * UNITS_A * 16
CAP = UNITS_A * 16 + 16
CH = 4


def _sc_body(mem_h, x_h, keys_h, vals_h, lab_h, y_h, out_h, olab_h,
             src_v, wj_v, keys_v, vals_v, lrows, lsrcs,
             stg0, stg1, y_v, lab_v, olab_v,
             sem_c, sem_g, sem_s, sem_s2):
    wid = lax.axis_index("s") * 2 + lax.axis_index("c")
    lanes = lax.iota(jnp.int32, 16)
    stg = (stg0, stg1)
    gsem = (sem_g, sem_s)

    def work(units, base):
        n = units * 16
        pltpu.sync_copy(keys_h, keys_v)
        pltpu.sync_copy(vals_h, vals_v)
        pltpu.sync_copy(y_h, y_v)
        pltpu.sync_copy(lab_h.at[pl.ds(base, n)], lab_v.at[pl.ds(0, n)])

        neg1 = jnp.zeros((16,), jnp.int32) - 1
        sentB = jnp.zeros((16,), jnp.int32) + B
        for i in range(units):
            wj_v[pl.ds(i * 16, 16)] = neg1
            src_v[pl.ds(i * 16, 16)] = sentB

        def scan(t, carry):
            k16 = keys_v[pl.ds(t * 16, 16)]
            v16 = vals_v[pl.ds(t * 16, 16)]
            j16 = lanes + t * 16
            loc = k16 - base
            inr = (loc >= 0) & (loc < n)
            locc = jnp.minimum(jnp.maximum(loc, 0), n - 1)
            curj = plsc.load_gather(wj_v, [locc])
            upd = inr & (j16 > curj)
            plsc.store_scatter(wj_v, [locc], j16, mask=upd)
            plsc.store_scatter(src_v, [locc], v16, mask=upd)
            return carry

        lax.fori_loop(0, B // 16, scan, jnp.int32(0))

        cur = jnp.int32(0)
        for i in range(units):
            s16 = src_v[pl.ds(i * 16, 16)]
            ovw = s16 < B
            rows16 = lanes + (base + i * 16)
            csum = plsc.cumsum(ovw.astype(jnp.int32))
            pos16 = cur + csum - 1
            plsc.store_scatter(lrows, [pos16], rows16, mask=ovw)
            plsc.store_scatter(lsrcs, [pos16], s16, mask=ovw)
            cur = cur + csum[15]
            g = plsc.load_gather(y_v, [jnp.minimum(s16, B - 1)])
            olab_v[pl.ds(i * 16, 16)] = jnp.where(ovw, g, lab_v[pl.ds(i * 16, 16)])
        pltpu.sync_copy(olab_v.at[pl.ds(0, n)], olab_h.at[pl.ds(base, n)])

        @pl.when(cur > 0)
        def _pad():
            zero16 = jnp.zeros((16,), jnp.int32)
            r0 = plsc.load_gather(lrows, [zero16])
            s0 = plsc.load_gather(lsrcs, [zero16])
            lrows[pl.ds(cur, 16)] = r0
            lsrcs[pl.ds(cur, 16)] = s0

        nc_copy = units * 16 // CH
        stores = [None, None]
        for u in range(nc_copy):
            b = u % 2
            if u >= 2:
                stores[b].wait()
            pltpu.sync_copy(mem_h.at[pl.ds(base + u * CH, CH)], stg[b])
            st = pltpu.make_async_copy(
                stg[b], out_h.at[pl.ds(base + u * CH, CH)], sem_c if b == 0 else sem_s2)
            st.start()
            stores[b] = st
        stores[0].wait()
        stores[1].wait()

        nc = (cur + 15) // 16

        def group(c, carry):
            keys16 = lrows[pl.ds(c * 16, 16)]
            srcs16 = lsrcs[pl.ds(c * 16, 16)]
            for half in range(2):
                for b in range(2):
                    for k in range(4):
                        i = half * 8 + b * 4 + k
                        pltpu.make_async_copy(
                            x_h.at[pl.ds(srcs16[i], 1)],
                            stg[b].at[pl.ds(k, 1)], gsem[b]).start()
                for b in range(2):
                    for k in range(4):
                        i = half * 8 + b * 4 + k
                        pltpu.make_async_copy(
                            x_h.at[pl.ds(srcs16[i], 1)],
                            stg[b].at[pl.ds(k, 1)], gsem[b]).wait()
                    for k in range(4):
                        i = half * 8 + b * 4 + k
                        pltpu.make_async_copy(
                            stg[b].at[pl.ds(k, 1)],
                            out_h.at[pl.ds(keys16[i], 1)], gsem[b]).start()
                for b in range(2):
                    for k in range(4):
                        i = half * 8 + b * 4 + k
                        pltpu.make_async_copy(
                            stg[b].at[pl.ds(k, 1)],
                            out_h.at[pl.ds(keys16[i], 1)], gsem[b]).wait()
            return carry

        lax.fori_loop(0, nc, group, jnp.int32(0))

    @pl.when(wid < SPLIT)
    def _a():
        work(UNITS_A, wid * (UNITS_A * 16))

    @pl.when(wid >= SPLIT)
    def _b():
        work(UNITS_B, BASE_B + (wid - SPLIT) * (UNITS_B * 16))


def kernel(mem, buffer_label, idx_keys, idx_vals, x, y):
    mesh = plsc.VectorSubcoreMesh(core_axis_name="c", subcore_axis_name="s")
    out, olab = pl.kernel(
        _sc_body,
        mesh=mesh,
        compiler_params=pltpu.CompilerParams(needs_layout_passes=False),
        out_type=[
            jax.ShapeDtypeStruct((M,) + IMG, jnp.float32),
            jax.ShapeDtypeStruct((M,), buffer_label.dtype),
        ],
        scratch_types=[
            pltpu.VMEM((UNITS_A * 16,), jnp.int32),
            pltpu.VMEM((UNITS_A * 16,), jnp.int32),
            pltpu.VMEM((B,), jnp.int32),
            pltpu.VMEM((B,), jnp.int32),
            pltpu.VMEM((CAP,), jnp.int32),
            pltpu.VMEM((CAP,), jnp.int32),
            pltpu.VMEM((CH,) + IMG, jnp.float32),
            pltpu.VMEM((CH,) + IMG, jnp.float32),
            pltpu.VMEM((B,), jnp.int32),
            pltpu.VMEM((UNITS_A * 16,), jnp.int32),
            pltpu.VMEM((UNITS_A * 16,), jnp.int32),
            pltpu.SemaphoreType.DMA,
            pltpu.SemaphoreType.DMA,
            pltpu.SemaphoreType.DMA,
            pltpu.SemaphoreType.DMA,
        ],
    )(mem, x, idx_keys, idx_vals, buffer_label, y)

    return out, olab


# speedup vs baseline: 4.4245x; 4.4245x over previous
import jax
import jax.numpy as jnp
from jax import lax
from jax.experimental import pallas as pl
from jax.experimental.pallas import tpu as pltpu
from jax.experimental.pallas import tpu_sc as plsc

M = 10000
B = 4096
ROW = 3072

NW = 32
UNITS_A = 20
UNITS_B = 19
SPLIT = 17
BASE_B = SPLIT * UNITS_A * 16
CAP = UNITS_A * 16 + 16


def _sc_body(out_h, olab_h, x_h, keys_h, vals_h, y_h,
             src_v, wj_v, keys_v, vals_v, lrows, lsrcs, kbuf, sbuf,
             kbuf2, sbuf2, rowbuf, rowbuf2, y_v, lab_v, olab_v,
             sem_g, sem_s, sem_g2, sem_s2):
    wid = lax.axis_index("s") * 2 + lax.axis_index("c")
    lanes = lax.iota(jnp.int32, 16)

    def work(units, base):
        n = units * 16
        pltpu.sync_copy(keys_h, keys_v)
        pltpu.sync_copy(vals_h, vals_v)
        pltpu.sync_copy(y_h, y_v)
        pltpu.sync_copy(olab_h.at[pl.ds(base, n)], lab_v.at[pl.ds(0, n)])

        neg1 = jnp.zeros((16,), jnp.int32) - 1
        sentB = jnp.zeros((16,), jnp.int32) + B
        for i in range(units):
            wj_v[pl.ds(i * 16, 16)] = neg1
            src_v[pl.ds(i * 16, 16)] = sentB

        def scan(t, carry):
            k16 = keys_v[pl.ds(t * 16, 16)]
            v16 = vals_v[pl.ds(t * 16, 16)]
            j16 = lanes + t * 16
            loc = k16 - base
            inr = (loc >= 0) & (loc < n)
            locc = jnp.minimum(jnp.maximum(loc, 0), n - 1)
            curj = plsc.load_gather(wj_v, [locc])
            upd = inr & (j16 > curj)
            plsc.store_scatter(wj_v, [locc], j16, mask=upd)
            plsc.store_scatter(src_v, [locc], v16, mask=upd)
            return carry

        lax.fori_loop(0, B // 16, scan, jnp.int32(0))

        cur = jnp.int32(0)
        for i in range(units):
            s16 = src_v[pl.ds(i * 16, 16)]
            ovw = s16 < B
            rows16 = lanes + (base + i * 16)
            csum = plsc.cumsum(ovw.astype(jnp.int32))
            pos16 = cur + csum - 1
            plsc.store_scatter(lrows, [pos16], rows16, mask=ovw)
            plsc.store_scatter(lsrcs, [pos16], s16, mask=ovw)
            cur = cur + csum[15]
            g = plsc.load_gather(y_v, [jnp.minimum(s16, B - 1)])
            olab_v[pl.ds(i * 16, 16)] = jnp.where(ovw, g, lab_v[pl.ds(i * 16, 16)])
        pltpu.sync_copy(olab_v.at[pl.ds(0, n)], olab_h.at[pl.ds(base, n)])

        @pl.when(cur > 0)
        def _pad():
            zero16 = jnp.zeros((16,), jnp.int32)
            r0 = plsc.load_gather(lrows, [zero16])
            s0 = plsc.load_gather(lsrcs, [zero16])
            lrows[pl.ds(cur, 16)] = r0
            lsrcs[pl.ds(cur, 16)] = s0

        nc = (cur + 15) // 16

        def start_gather(c, kb, sb, gsem):
            kb[...] = lrows[pl.ds(c * 16, 16)]
            sb[...] = lsrcs[pl.ds(c * 16, 16)]
            cp = pltpu.make_async_copy(
                x_h.at[sb], rowbuf if kb is kbuf else rowbuf2, gsem)
            cp.start()
            return cp

        @pl.when(nc > 0)
        def _scatter():
            g0 = start_gather(jnp.int32(0), kbuf, sbuf, sem_g)
            g0.wait()

            def chunk(c, carry):
                even = c % 2 == 0
                @pl.when(c + 1 < nc)
                def _next():
                    @pl.when(even)
                    def _():
                        start_gather(c + 1, kbuf2, sbuf2, sem_g2)
                    @pl.when(jnp.logical_not(even))
                    def _():
                        start_gather(c + 1, kbuf, sbuf, sem_g)
                @pl.when(even)
                def _():
                    pltpu.async_copy(rowbuf, out_h.at[kbuf], sem_s).wait()
                @pl.when(jnp.logical_not(even))
                def _():
                    pltpu.async_copy(rowbuf2, out_h.at[kbuf2], sem_s2).wait()
                @pl.when(c + 1 < nc)
                def _drain():
                    @pl.when(even)
                    def _():
                        pltpu.make_async_copy(x_h.at[sbuf2], rowbuf2, sem_g2).wait()
                    @pl.when(jnp.logical_not(even))
                    def _():
                        pltpu.make_async_copy(x_h.at[sbuf], rowbuf, sem_g).wait()
                return carry

            lax.fori_loop(0, nc, chunk, jnp.int32(0))

    @pl.when(wid < SPLIT)
    def _a():
        work(UNITS_A, wid * (UNITS_A * 16))

    @pl.when(wid >= SPLIT)
    def _b():
        work(UNITS_B, BASE_B + (wid - SPLIT) * (UNITS_B * 16))


def kernel(mem, buffer_label, idx_keys, idx_vals, x, y):
    mem2 = mem.reshape(M, ROW)
    x2 = x.reshape(B, ROW)

    out_ref = jax.new_ref(mem2)
    olab_ref = jax.new_ref(buffer_label)

    mesh = plsc.VectorSubcoreMesh(core_axis_name="c", subcore_axis_name="s")
    pl.kernel(
        _sc_body,
        mesh=mesh,
        compiler_params=pltpu.CompilerParams(needs_layout_passes=False),
        out_type=(),
        scratch_types=[
            pltpu.VMEM((UNITS_A * 16,), jnp.int32),
            pltpu.VMEM((UNITS_A * 16,), jnp.int32),
            pltpu.VMEM((B,), jnp.int32),
            pltpu.VMEM((B,), jnp.int32),
            pltpu.VMEM((CAP,), jnp.int32),
            pltpu.VMEM((CAP,), jnp.int32),
            pltpu.VMEM((16,), jnp.int32),
            pltpu.VMEM((16,), jnp.int32),
            pltpu.VMEM((16,), jnp.int32),
            pltpu.VMEM((16,), jnp.int32),
            pltpu.VMEM((16, ROW), jnp.float32),
            pltpu.VMEM((16, ROW), jnp.float32),
            pltpu.VMEM((B,), jnp.int32),
            pltpu.VMEM((UNITS_A * 16,), jnp.int32),
            pltpu.VMEM((UNITS_A * 16,), jnp.int32),
            pltpu.SemaphoreType.DMA,
            pltpu.SemaphoreType.DMA,
            pltpu.SemaphoreType.DMA,
            pltpu.SemaphoreType.DMA,
        ],
    )(out_ref, olab_ref, x2, idx_keys, idx_vals, y)

    return out_ref[...].reshape(mem.shape), olab_ref[...]
